# baseline (speedup 1.0000x reference)
import functools

import jax
import jax.numpy as jnp
from jax import lax
from jax.experimental import pallas as pl
from jax.experimental.pallas import tpu as pltpu
from jax.experimental.pallas import tpu_sc as plsc

STRENGTH = 0.9
THRESHOLD = 0.5

ROWS, COLS = 16384, 1024
NC, NS, L = 2, 16, 16
NW = NC * NS
RPW = ROWS // NW
CR = 32
CE = CR * COLS
NCHUNK = RPW // CR

_MESH = plsc.VectorSubcoreMesh(
    core_axis_name="c", subcore_axis_name="s", num_cores=NC, num_subcores=NS
)


@functools.partial(
    pl.kernel,
    out_type=jax.ShapeDtypeStruct((ROWS * COLS,), jnp.float32),
    mesh=_MESH,
    scratch_types=[
        pltpu.VMEM((CE,), jnp.float32),
        pltpu.VMEM((CE,), jnp.float32),
        pltpu.SemaphoreType.DMA,
        pltpu.SemaphoreType.DMA,
    ],
    compiler_params=pltpu.CompilerParams(needs_layout_passes=False),
)
def _sc_rewrite(z_hbm, out_hbm, buf0, buf1, sem0, sem1):
    wid = lax.axis_index("s") * NC + lax.axis_index("c")
    base = wid * (RPW * COLS)
    bufs = (buf0, buf1)
    sems = (sem0, sem1)

    def start_fetch(ci, b):
        pltpu.async_copy(z_hbm.at[pl.ds(base + ci * CE, CE)], bufs[b], sems[b])

    def wait_fetch(ci, b):
        pltpu.make_async_copy(
            z_hbm.at[pl.ds(base + ci * CE, CE)], bufs[b], sems[b]
        ).wait()

    lane = jnp.arange(L, dtype=jnp.int32)
    one16 = jnp.ones((L,), jnp.int32)

    start_fetch(0, 0)
    start_fetch(1, 1)

    def body(i, carry):
        for b in range(2):
            g = 2 * i + b
            wait_fetch(g, b)
            for h in range(CR // L):
                rid0 = (lane + h * L) * COLS
                vals = plsc.load_gather(bufs[b], [rid0])
                wet = STRENGTH / (1.0 + jnp.exp((THRESHOLD - vals) * 10.0))
                plsc.store_scatter(bufs[b], [rid0 + one16], wet)
            pltpu.sync_copy(bufs[b], out_hbm.at[pl.ds(base + g * CE, CE)])

            @pl.when(g + 2 < NCHUNK)
            def _():
                start_fetch(g + 2, b)

        return carry

    lax.fori_loop(0, NCHUNK // 2, body, 0)


def kernel(z):
    return _sc_rewrite(z.reshape(-1)).reshape(ROWS, COLS)


# speedup vs baseline: 1.5357x; 1.5357x over previous
import functools

import jax
import jax.numpy as jnp
from jax import lax
from jax.experimental import pallas as pl
from jax.experimental.pallas import tpu as pltpu
from jax.experimental.pallas import tpu_sc as plsc

STRENGTH = 0.9
THRESHOLD = 0.5

ROWS, COLS = 16384, 1024
NC, NS, L = 2, 16, 16
NW = NC * NS
SC_ROWS = 512
RPW = SC_ROWS // NW
CE = RPW * COLS

TC_BR = 512

_MESH = plsc.VectorSubcoreMesh(
    core_axis_name="c", subcore_axis_name="s", num_cores=NC, num_subcores=NS
)


@functools.partial(
    pl.kernel,
    out_type=jax.ShapeDtypeStruct((SC_ROWS * COLS,), jnp.float32),
    mesh=_MESH,
    scratch_types=[
        pltpu.VMEM((CE,), jnp.float32),
        pltpu.SemaphoreType.DMA,
    ],
    compiler_params=pltpu.CompilerParams(needs_layout_passes=False),
)
def _sc_tile(z_hbm, t_hbm, buf, sem):
    wid = lax.axis_index("s") * NC + lax.axis_index("c")
    base = wid * CE
    pltpu.async_copy(z_hbm.at[pl.ds(base, CE)], buf, sem).wait()
    rid0 = jnp.arange(L, dtype=jnp.int32) * COLS
    vals = plsc.load_gather(buf, [rid0])
    wet = STRENGTH / (1.0 + jnp.exp((THRESHOLD - vals) * 10.0))
    plsc.store_scatter(buf, [rid0 + jnp.ones((L,), jnp.int32)], wet)
    pltpu.sync_copy(buf, t_hbm.at[pl.ds(base, CE)])


def _bulk_body(z_ref, o_ref):
    zb = z_ref[...]
    wet = jax.nn.sigmoid((zb[:, 0:1] - THRESHOLD) * 10.0) * STRENGTH
    lane = lax.broadcasted_iota(jnp.int32, zb.shape, 1)
    o_ref[...] = jnp.where(lane == 1, wet, zb)


def _tc_bulk(z):
    grid = ((ROWS - SC_ROWS) // TC_BR,)
    return pl.pallas_call(
        _bulk_body,
        grid=grid,
        in_specs=[pl.BlockSpec((TC_BR, COLS), lambda i: (i + 1, 0))],
        out_specs=pl.BlockSpec((TC_BR, COLS), lambda i: (i + 1, 0)),
        out_shape=jax.ShapeDtypeStruct((ROWS, COLS), jnp.float32),
    )(z)


def _merge_body(t_ref, a_ref, o_ref):
    o_ref[...] = t_ref[...]


def _tc_merge(t, a):
    return pl.pallas_call(
        _merge_body,
        grid=(1,),
        in_specs=[
            pl.BlockSpec((SC_ROWS, COLS), lambda i: (0, 0)),
            pl.BlockSpec(memory_space=pltpu.MemorySpace.HBM),
        ],
        out_specs=pl.BlockSpec((SC_ROWS, COLS), lambda i: (0, 0)),
        out_shape=jax.ShapeDtypeStruct((ROWS, COLS), jnp.float32),
        input_output_aliases={1: 0},
    )(t, a)


def kernel(z):
    t = _sc_tile(z.reshape(-1))
    a = _tc_bulk(z)
    return _tc_merge(t.reshape(SC_ROWS, COLS), a)


# speedup vs baseline: 2.7783x; 1.8091x over previous
import functools

import jax
import jax.numpy as jnp
from jax import lax
from jax.experimental import pallas as pl
from jax.experimental.pallas import tpu as pltpu
from jax.experimental.pallas import tpu_sc as plsc

STRENGTH = 0.9
THRESHOLD = 0.5

ROWS, COLS = 16384, 1024
NC, NS, L = 2, 16, 16
NW = NC * NS
SC_ROWS = 512
RPW = SC_ROWS // NW
CE = RPW * COLS

TC_BR = 512

_MESH = plsc.VectorSubcoreMesh(
    core_axis_name="c", subcore_axis_name="s", num_cores=NC, num_subcores=NS
)


@functools.partial(
    pl.kernel,
    out_type=jax.ShapeDtypeStruct((SC_ROWS, COLS), jnp.float32),
    mesh=_MESH,
    scratch_types=[
        pltpu.VMEM((RPW, COLS), jnp.float32),
        pltpu.SemaphoreType.DMA,
    ],
    compiler_params=pltpu.CompilerParams(needs_layout_passes=False),
)
def _sc_tile(z_hbm, t_hbm, buf, sem):
    wid = lax.axis_index("s") * NC + lax.axis_index("c")
    base = wid * RPW
    pltpu.async_copy(z_hbm.at[pl.ds(base, RPW)], buf, sem).wait()
    rid = jnp.arange(L, dtype=jnp.int32)
    zero16 = jnp.zeros((L,), jnp.int32)
    vals = plsc.load_gather(buf, [rid, zero16])
    wet = STRENGTH / (1.0 + jnp.exp((THRESHOLD - vals) * 10.0))
    plsc.store_scatter(buf, [rid, zero16 + 1], wet)
    pltpu.sync_copy(buf, t_hbm.at[pl.ds(base, RPW)])


def _bulk_body(z_ref, o_ref):
    zb = z_ref[...]
    wet = jax.nn.sigmoid((zb[:, 0:1] - THRESHOLD) * 10.0) * STRENGTH
    lane = lax.broadcasted_iota(jnp.int32, zb.shape, 1)
    o_ref[...] = jnp.where(lane == 1, wet, zb)


def _tc_bulk(z):
    grid = ((ROWS - SC_ROWS) // TC_BR,)
    return pl.pallas_call(
        _bulk_body,
        grid=grid,
        in_specs=[pl.BlockSpec((TC_BR, COLS), lambda i: (i + 1, 0))],
        out_specs=pl.BlockSpec((TC_BR, COLS), lambda i: (i + 1, 0)),
        out_shape=jax.ShapeDtypeStruct((ROWS, COLS), jnp.float32),
    )(z)


def _merge_body(t_ref, a_ref, o_ref):
    o_ref[...] = t_ref[...]


def _tc_merge(t, a):
    return pl.pallas_call(
        _merge_body,
        grid=(1,),
        in_specs=[
            pl.BlockSpec((SC_ROWS, COLS), lambda i: (0, 0)),
            pl.BlockSpec(memory_space=pltpu.MemorySpace.HBM),
        ],
        out_specs=pl.BlockSpec((SC_ROWS, COLS), lambda i: (0, 0)),
        out_shape=jax.ShapeDtypeStruct((ROWS, COLS), jnp.float32),
        input_output_aliases={1: 0},
    )(t, a)


def kernel(z):
    t = _sc_tile(z)
    a = _tc_bulk(z)
    return _tc_merge(t, a)


# speedup vs baseline: 3.8447x; 1.3838x over previous
import jax
import jax.numpy as jnp
from jax.experimental import pallas as pl

STRENGTH = 0.9
THRESHOLD = 0.5

ROWS, COLS = 16384, 1024
BLOCK_ROWS = 512


def _body(z_ref, o_ref):
    zb = z_ref[...]
    col0 = zb[:, 0:1]
    wet = jax.nn.sigmoid((col0 - THRESHOLD) * 10.0) * STRENGTH
    lane = jax.lax.broadcasted_iota(jnp.int32, zb.shape, 1)
    o_ref[...] = jnp.where(lane == 1, wet, zb)


def kernel(z):
    grid = (ROWS // BLOCK_ROWS,)
    return pl.pallas_call(
        _body,
        grid=grid,
        in_specs=[pl.BlockSpec((BLOCK_ROWS, COLS), lambda i: (i, 0))],
        out_specs=pl.BlockSpec((BLOCK_ROWS, COLS), lambda i: (i, 0)),
        out_shape=jax.ShapeDtypeStruct((ROWS, COLS), jnp.float32),
    )(z)


# speedup vs baseline: 4.3733x; 1.1375x over previous
import jax
import jax.numpy as jnp
from jax.experimental import pallas as pl

STRENGTH = 0.9
THRESHOLD = 0.5

ROWS, COLS = 16384, 1024
BLOCK_ROWS = 2048


def _body(z_ref, o_ref):
    zb = z_ref[...]
    col0 = zb[:, 0:1]
    wet = jax.nn.sigmoid((col0 - THRESHOLD) * 10.0) * STRENGTH
    lane = jax.lax.broadcasted_iota(jnp.int32, zb.shape, 1)
    o_ref[...] = jnp.where(lane == 1, wet, zb)


def kernel(z):
    grid = (ROWS // BLOCK_ROWS,)
    return pl.pallas_call(
        _body,
        grid=grid,
        in_specs=[pl.BlockSpec((BLOCK_ROWS, COLS), lambda i: (i, 0))],
        out_specs=pl.BlockSpec((BLOCK_ROWS, COLS), lambda i: (i, 0)),
        out_shape=jax.ShapeDtypeStruct((ROWS, COLS), jnp.float32),
    )(z)
